# baseline (speedup 1.0000x reference)
import functools

import jax
import jax.numpy as jnp
from jax import lax
from jax.experimental import pallas as pl
from jax.experimental.pallas import tpu as pltpu
from jax.experimental.pallas import tpu_sc as plsc

K = 20
P = 20

_NC = 2
_NS = 16
_NW = _NC * _NS


def _gather_max_body(k, C, G, M, table_hbm, gidx_hbm, out_hbm,
                     idx_v, rows_v, out_v, sem0, sem1):
    ppw = M // _NW
    n_groups = ppw // G
    WB = 64
    W = WB // G
    wid = lax.axis_index("s") * _NC + lax.axis_index("c")
    pt_base = wid * ppw

    pltpu.sync_copy(
        gidx_hbm.at[pl.ds(pl.multiple_of(wid * n_groups, 8), n_groups)],
        idx_v)

    sems = (sem0, sem1)

    def _start(g, b):
        pltpu.make_async_copy(
            table_hbm.at[idx_v.at[g]], rows_v.at[b], sems[b]).start()

    def _wait(b):
        pltpu.make_async_copy(
            table_hbm.at[idx_v.at[0]], rows_v.at[b], sems[b]).wait()

    _start(0, 0)
    _start(1, 1)

    def outer(i, _):
        for b in range(2):
            g = 2 * i + b
            _wait(b)
            rows_b = rows_v.at[b]
            for p in range(G):
                p_local = (g % W) * G + p
                for cc in range(C // 16):
                    sl = pl.ds(cc * 16, 16)
                    acc = rows_b[p * k, sl]

                    def jbody(j, a):
                        return jnp.maximum(a, rows_b[p * k + j, sl])

                    acc = lax.fori_loop(1, k, jbody, acc, unroll=4)
                    out_v[p_local, sl] = acc

            @pl.when(g + 2 < n_groups)
            def _():
                _start(g + 2, b)

            @pl.when(g % W == W - 1)
            def _():
                off = (g + 1 - W) * G
                pltpu.sync_copy(
                    out_v,
                    out_hbm.at[pl.ds(pl.multiple_of(pt_base + off, 8), WB)])
        return _

    lax.fori_loop(0, n_groups // 2, outer, None)


def _sc_gather_max(table, gidx, k):
    M, C = table.shape
    G = max(1, 120 // k)
    while (M // _NW) % G:
        G -= 1
    n_groups = (M // _NW) // G
    gidx2 = gidx.reshape(M // G, G * k)
    mesh = plsc.VectorSubcoreMesh(core_axis_name="c", subcore_axis_name="s")
    body = functools.partial(_gather_max_body, k, C, G, M)
    fn = pl.kernel(
        body,
        out_type=jax.ShapeDtypeStruct((M, C), jnp.float32),
        mesh=mesh,
        scratch_types=[
            pltpu.VMEM((n_groups, G * k), jnp.int32),
            pltpu.VMEM((2, G * k, C), jnp.float32),
            pltpu.VMEM((64, C), jnp.float32),
            pltpu.SemaphoreType.DMA,
            pltpu.SemaphoreType.DMA,
        ],
        compiler_params=pltpu.CompilerParams(use_tc_tiling_on_sc=False),
    )
    return fn(table, gidx2)


def _gn_lrelu(h, g, b, G):
    C, N = h.shape
    hg = h.reshape(G, C // G, N)
    m = jnp.mean(hg, axis=(1, 2), keepdims=True)
    v = jnp.mean((hg - m) ** 2, axis=(1, 2), keepdims=True)
    hg = (hg - m) * lax.rsqrt(v + 1e-5)
    h = hg.reshape(C, N)
    h = h * g + b
    return jnp.where(h >= 0, h, 0.2 * h)


def _l5_kernel(xc_ref, Wa_ref, ga_ref, ba_ref, Wb_ref, gb_ref, bb_ref, out_ref):
    xc = xc_ref[0]
    h = jnp.dot(Wa_ref[...], xc, preferred_element_type=jnp.float32)
    h = _gn_lrelu(h, ga_ref[...], ba_ref[...], 16)
    h2 = jnp.dot(Wb_ref[...], h, preferred_element_type=jnp.float32)
    h2 = _gn_lrelu(h2, gb_ref[...], bb_ref[...], 16)
    out_ref[0] = h2


def _layer5(xc, W5a, g5a, b5a, W5b, g5b, b5b):
    B, C, N = xc.shape
    return pl.pallas_call(
        _l5_kernel,
        grid=(B,),
        in_specs=[
            pl.BlockSpec((1, C, N), lambda b: (b, 0, 0)),
            pl.BlockSpec((1024, 512), lambda b: (0, 0)),
            pl.BlockSpec((1024, 1), lambda b: (0, 0)),
            pl.BlockSpec((1024, 1), lambda b: (0, 0)),
            pl.BlockSpec((512, 1024), lambda b: (0, 0)),
            pl.BlockSpec((512, 1), lambda b: (0, 0)),
            pl.BlockSpec((512, 1), lambda b: (0, 0)),
        ],
        out_specs=pl.BlockSpec((1, 512, N), lambda b: (b, 0, 0)),
        out_shape=jax.ShapeDtypeStruct((B, 512, N), jnp.float32),
    )(xc, W5a, g5a.reshape(-1, 1), b5a.reshape(-1, 1),
      W5b, g5b.reshape(-1, 1), b5b.reshape(-1, 1))


def _conv(x, W):
    return jnp.einsum('oc,bcn->bon', W, x)


def _gn(x, gamma, beta, G, eps=1e-5):
    B, C, N = x.shape
    xg = x.reshape(B, G, C // G, N)
    m = jnp.mean(xg, axis=(2, 3), keepdims=True)
    v = jnp.var(xg, axis=(2, 3), keepdims=True)
    xg = (xg - m) / jnp.sqrt(v + eps)
    x = xg.reshape(B, C, N)
    return x * gamma[None, :, None] + beta[None, :, None]


def _lrelu(x):
    return jnp.where(x >= 0, x, 0.2 * x)


def _block(x, W, g, b, G):
    return _lrelu(_gn(_conv(x, W), g, b, G))


def _knn(x, k):
    inner = -2.0 * jnp.einsum('bcn,bcm->bnm', x, x)
    xx = jnp.sum(x ** 2, axis=1, keepdims=True)
    pd = -xx - inner - jnp.transpose(xx, (0, 2, 1))
    _, idx = jax.lax.top_k(pd, k)
    return idx


def _point_conv(x_in, Wa, ga, ba, Wb, gb, bb, G, gidx, k):
    fa = _block(x_in, Wa, ga, ba, G)
    fb = _block(x_in, Wb, gb, bb, G)
    B, C, N = fa.shape
    table = fa.transpose(0, 2, 1).reshape(B * N, C)
    agg = _sc_gather_max(table, gidx[:, :k], k)
    agg = agg.reshape(B, N, C).transpose(0, 2, 1)
    return agg + fb


def kernel(x, W1a, g1a, b1a, W1b, g1b, b1b, W2a, g2a, b2a, W2b, g2b, b2b, W3a, g3a, b3a, W3b, g3b, b3b, W4a, g4a, b4a, W4b, g4b, b4b, W5a, g5a, b5a, W5b, g5b, b5b):
    xt = jnp.transpose(x, (0, 2, 1))
    B, _, N = xt.shape
    pool_size = K + 3 * P
    idx_pool = _knn(xt, pool_size)
    gidx = (idx_pool + (jnp.arange(B, dtype=jnp.int32) * N)[:, None, None])
    gidx = gidx.reshape(B * N, pool_size)
    x1 = _point_conv(xt, W1a, g1a, b1a, W1b, g1b, b1b, 8, gidx, K)
    x2 = _point_conv(x1, W2a, g2a, b2a, W2b, g2b, b2b, 8, gidx, K + P)
    x3 = _point_conv(x2, W3a, g3a, b3a, W3b, g3b, b3b, 8, gidx, K + 2 * P)
    x4 = _point_conv(x3, W4a, g4a, b4a, W4b, g4b, b4b, 16, gidx, K + 3 * P)
    xc = jnp.concatenate((x1, x2, x3, x4), axis=1)
    x6 = _layer5(xc, W5a, g5a, b5a, W5b, g5b, b5b)
    return jnp.transpose(x6, (0, 2, 1))
